# speedup vs baseline: 1.5157x; 1.5157x over previous
import functools

import jax
import jax.numpy as jnp
import numpy as np
from jax import lax
from jax.experimental import pallas as pl

_TAU = 0.5
_K = 10
_TINY = float(np.finfo(np.float32).tiny)
_GUMBEL_SHIFT = 34.0


def _rotl(x, r):
    return (x << jnp.uint32(r)) | (x >> jnp.uint32(32 - r))


def _threefry_bits(c1):
    ks0 = jnp.uint32(0)
    ks1 = jnp.uint32(1)
    ks2 = jnp.uint32(0x1BD11BDB)
    rot_a = (13, 15, 26, 6)
    rot_b = (17, 29, 16, 24)
    injections = ((ks1, ks2), (ks2, ks0), (ks0, ks1), (ks1, ks2), (ks2, ks0))
    x0 = ks0
    x1 = c1 + ks1
    for i, rots in enumerate((rot_a, rot_b, rot_a, rot_b, rot_a)):
        for r in rots:
            x0 = x0 + x1
            x1 = _rotl(x1, r)
            x1 = x0 ^ x1
        x0 = x0 + injections[i][0]
        x1 = x1 + injections[i][1] + jnp.uint32(i + 1)
    return x0 ^ x1


def _body(logits_ref, out_ref, *, block_rows, d):
    logits = logits_ref[:]
    row_max = jnp.max(logits, axis=1, keepdims=True)
    e0 = jnp.exp(2.0 * (logits - row_max) - _GUMBEL_SHIFT)

    step = pl.program_id(0)
    row = lax.broadcasted_iota(jnp.uint32, (block_rows, d), 0)
    col = lax.broadcasted_iota(jnp.uint32, (block_rows, d), 1)
    b = row + jnp.uint32(block_rows) * step.astype(jnp.uint32)
    base = (b * jnp.uint32(_K)) * jnp.uint32(d) + col

    acc = jnp.zeros((block_rows, d), jnp.float32)
    for k in range(_K):
        bits = _threefry_bits(base + jnp.uint32(k * d))
        fbits = (bits >> jnp.uint32(9)) | jnp.uint32(0x3F800000)
        frac = lax.bitcast_convert_type(fbits, jnp.float32) - 1.0
        u = jnp.maximum(jnp.float32(_TINY), frac + jnp.float32(_TINY))
        lu = -jnp.log(u)
        e = e0 / (lu * lu)
        s = jnp.sum(e, axis=1, keepdims=True)
        acc = jnp.maximum(acc, e / s)
    out_ref[:] = acc


@jax.jit
def kernel(logits):
    bsz, d = logits.shape
    block_rows = 8
    grid = bsz // block_rows
    return pl.pallas_call(
        functools.partial(_body, block_rows=block_rows, d=d),
        grid=(grid,),
        in_specs=[pl.BlockSpec((block_rows, d), lambda i: (i, 0))],
        out_specs=pl.BlockSpec((block_rows, d), lambda i: (i, 0)),
        out_shape=jax.ShapeDtypeStruct((bsz, d), jnp.float32),
    )(logits)
